# Initial kernel scaffold; baseline (speedup 1.0000x reference)
import jax
import jax.numpy as jnp
from jax.experimental import pallas as pl


def kernel(X, emb_E, emb_E_var, emb_R, emb_R_var, emb_TE, alpha_E, beta_E, omega_E, emb_TR, alpha_R, beta_R, omega_R):
    raise NotImplementedError("write your pallas kernel here")


# speedup vs baseline: 1.0449x; 1.0449x over previous
import functools
import jax
import jax.numpy as jnp
from jax import lax
from jax.experimental import pallas as pl
from jax.experimental.pallas import tpu as pltpu
from jax.experimental.pallas import tpu_sc as plsc

D = 64
L = 16
TWO_PI = 6.283185307179586


def _sin2pi(x):
    u = lax.rem(x, jnp.float32(1.0))
    u = jnp.where(u > 0.5, u - 1.0, u)
    u = jnp.where(u < -0.5, u + 1.0, u)
    u = jnp.where(u > 0.25, 0.5 - u, u)
    u = jnp.where(u < -0.25, -0.5 - u, u)
    th = jnp.float32(TWO_PI) * u
    t2 = th * th
    p = jnp.float32(2.7557319e-06)
    p = p * t2 - jnp.float32(1.9841270e-04)
    p = p * t2 + jnp.float32(8.3333333e-03)
    p = p * t2 - jnp.float32(0.16666667)
    p = p * t2 + jnp.float32(1.0)
    return th * p


def kernel(X, emb_E, emb_E_var, emb_R, emb_R_var, emb_TE, alpha_E, beta_E,
           omega_E, emb_TR, alpha_R, beta_R, omega_R):
    B = X.shape[0]
    h_i = X[:, 0]
    t_i = X[:, 1]
    r_i = X[:, 2]
    d_f = X[:, 3].astype(jnp.float32)
    alpha_E1 = alpha_E.reshape(-1)
    alpha_R1 = alpha_R.reshape(-1)

    info = plsc.get_sparse_core_info()
    NC, NS = info.num_cores, info.num_subcores
    NW = NC * NS
    C = 128
    rows_per_w = B // NW
    n_chunks = rows_per_w // C

    mesh = plsc.VectorSubcoreMesh(core_axis_name="c", subcore_axis_name="s")

    vm_rows = lambda: pltpu.VMEM((C, D), jnp.float32)

    @functools.partial(
        pl.kernel,
        out_type=jax.ShapeDtypeStruct((B,), jnp.float32),
        mesh=mesh,
        compiler_params=pltpu.CompilerParams(
            needs_layout_passes=False, use_tc_tiling_on_sc=False),
        scratch_types=[
            pltpu.VMEM((C,), jnp.int32),
            pltpu.VMEM((C,), jnp.int32),
            pltpu.VMEM((C,), jnp.int32),
            pltpu.VMEM((C,), jnp.float32),
            vm_rows(), vm_rows(), vm_rows(), vm_rows(), vm_rows(),
            vm_rows(), vm_rows(), vm_rows(), vm_rows(), vm_rows(),
            vm_rows(), vm_rows(), vm_rows(), vm_rows(), vm_rows(),
            pltpu.VMEM((C,), jnp.float32),
            pltpu.VMEM((C,), jnp.float32),
            pltpu.VMEM((C,), jnp.float32),
            pltpu.VMEM((C,), jnp.float32),
            pltpu.SemaphoreType.DMA,
        ],
    )
    def score_kernel(h_hbm, t_hbm, r_hbm, d_hbm,
                     eE, vE, eR, vR, eTE, aE, bE, oE, eTR, aR, bR, oR,
                     out_hbm,
                     hix, tix, rix, dvb,
                     heE, heTE, hbE, hoE, hvE,
                     teE, teTE, tbE, toE, tvE,
                     reR, reTR, rbR, roR, rvR,
                     hal, tal, ral,
                     outb, sem):
        wid = lax.axis_index("s") * NC + lax.axis_index("c")

        def do_chunk(ci, carry):
            base = pl.multiple_of(wid * rows_per_w + ci * C, C)
            pltpu.sync_copy(h_hbm.at[pl.ds(base, C)], hix)
            pltpu.sync_copy(t_hbm.at[pl.ds(base, C)], tix)
            pltpu.sync_copy(r_hbm.at[pl.ds(base, C)], rix)
            pltpu.sync_copy(d_hbm.at[pl.ds(base, C)], dvb)
            cps = [
                pltpu.async_copy(eE.at[hix], heE, sem),
                pltpu.async_copy(eTE.at[hix], heTE, sem),
                pltpu.async_copy(bE.at[hix], hbE, sem),
                pltpu.async_copy(oE.at[hix], hoE, sem),
                pltpu.async_copy(vE.at[hix], hvE, sem),
                pltpu.async_copy(aE.at[hix], hal, sem),
                pltpu.async_copy(eE.at[tix], teE, sem),
                pltpu.async_copy(eTE.at[tix], teTE, sem),
                pltpu.async_copy(bE.at[tix], tbE, sem),
                pltpu.async_copy(oE.at[tix], toE, sem),
                pltpu.async_copy(vE.at[tix], tvE, sem),
                pltpu.async_copy(aE.at[tix], tal, sem),
                pltpu.async_copy(eR.at[rix], reR, sem),
                pltpu.async_copy(eTR.at[rix], reTR, sem),
                pltpu.async_copy(bR.at[rix], rbR, sem),
                pltpu.async_copy(oR.at[rix], roR, sem),
                pltpu.async_copy(vR.at[rix], rvR, sem),
                pltpu.async_copy(aR.at[rix], ral, sem),
            ]
            for cp in cps:
                cp.wait()

            for g in range(C // L):
                rows = lax.iota(jnp.int32, L) + jnp.int32(g * L)
                d16 = dvb[pl.ds(g * L, L)]
                dah = d16 * hal[pl.ds(g * L, L)]
                dat = d16 * tal[pl.ds(g * L, L)]
                dar = d16 * ral[pl.ds(g * L, L)]

                def jbody(j, acc):
                    jv = jnp.full((L,), j, jnp.int32)
                    ld = lambda ref: plsc.load_gather(ref, [rows, jv])
                    hm = ld(heE) + dah * ld(heTE) + ld(hbE) * _sin2pi(ld(hoE) * d16)
                    tm = ld(teE) + dat * ld(teTE) + ld(tbE) * _sin2pi(ld(toE) * d16)
                    rm = ld(reR) + dar * ld(reTR) + ld(rbR) * _sin2pi(ld(roR) * d16)
                    m = hm - tm - rm
                    m2 = m * m
                    s = ld(hvE) + ld(tvE)
                    rv = ld(rvR)
                    num = s * (s + m2) + rv * (rv + m2)
                    return acc + num / (rv * s)

                acc = lax.fori_loop(0, D, jbody, jnp.zeros((L,), jnp.float32))
                outb[pl.ds(g * L, L)] = (acc - jnp.float32(2 * D)) * jnp.float32(0.25)

            pltpu.sync_copy(outb, out_hbm.at[pl.ds(base, C)])
            return carry

        lax.fori_loop(0, n_chunks, do_chunk, 0)

    return score_kernel(h_i, t_i, r_i, d_f,
                        emb_E, emb_E_var, emb_R, emb_R_var, emb_TE, alpha_E1,
                        beta_E, omega_E, emb_TR, alpha_R1, beta_R, omega_R)


# speedup vs baseline: 1.0550x; 1.0097x over previous
import functools
import jax
import jax.numpy as jnp
from jax import lax
from jax.experimental import pallas as pl
from jax.experimental.pallas import tpu as pltpu
from jax.experimental.pallas import tpu_sc as plsc

D = 64
L = 16
TWO_PI = 6.283185307179586


def _sin2pi(x):
    u = lax.rem(x, jnp.float32(1.0))
    u = jnp.where(u > 0.5, u - 1.0, u)
    u = jnp.where(u < -0.5, u + 1.0, u)
    u = jnp.where(u > 0.25, 0.5 - u, u)
    u = jnp.where(u < -0.25, -0.5 - u, u)
    th = jnp.float32(TWO_PI) * u
    t2 = th * th
    p = jnp.float32(2.7557319e-06)
    p = p * t2 - jnp.float32(1.9841270e-04)
    p = p * t2 + jnp.float32(8.3333333e-03)
    p = p * t2 - jnp.float32(0.16666667)
    p = p * t2 + jnp.float32(1.0)
    return th * p


def kernel(X, emb_E, emb_E_var, emb_R, emb_R_var, emb_TE, alpha_E, beta_E,
           omega_E, emb_TR, alpha_R, beta_R, omega_R):
    B = X.shape[0]
    h_i = X[:, 0]
    t_i = X[:, 1]
    r_i = X[:, 2]
    d_f = X[:, 3].astype(jnp.float32)
    alpha_E1 = alpha_E.reshape(-1)
    alpha_R1 = alpha_R.reshape(-1)

    info = plsc.get_sparse_core_info()
    NC, NS = info.num_cores, info.num_subcores
    NW = NC * NS
    C = 64
    rows_per_w = B // NW
    n_chunks = rows_per_w // C
    JU = 4

    mesh = plsc.VectorSubcoreMesh(core_axis_name="c", subcore_axis_name="s")

    big = lambda: pltpu.VMEM((C, D), jnp.float32)
    bigset = lambda: [big() for _ in range(15)]

    @functools.partial(
        pl.kernel,
        out_type=jax.ShapeDtypeStruct((B,), jnp.float32),
        mesh=mesh,
        compiler_params=pltpu.CompilerParams(
            needs_layout_passes=False, use_tc_tiling_on_sc=False),
        scratch_types=[
            pltpu.VMEM((rows_per_w,), jnp.int32),
            pltpu.VMEM((rows_per_w,), jnp.int32),
            pltpu.VMEM((rows_per_w,), jnp.int32),
            pltpu.VMEM((rows_per_w,), jnp.float32),
            pltpu.VMEM((rows_per_w,), jnp.float32),
            bigset(), bigset(),
            [pltpu.VMEM((C,), jnp.float32) for _ in range(3)],
            [pltpu.VMEM((C,), jnp.float32) for _ in range(3)],
            pltpu.SemaphoreType.DMA,
            pltpu.SemaphoreType.DMA,
            pltpu.SemaphoreType.DMA,
        ],
    )
    def score_kernel(h_hbm, t_hbm, r_hbm, d_hbm,
                     eE, vE, eR, vR, eTE, aE, bE, oE, eTR, aR, bR, oR,
                     out_hbm,
                     hix, tix, rix, dvb, outb,
                     set0, set1, al0, al1,
                     sem_i, sem0, sem1):
        wid = lax.axis_index("s") * NC + lax.axis_index("c")
        wbase = pl.multiple_of(wid * rows_per_w, rows_per_w)

        cps_i = [
            pltpu.async_copy(h_hbm.at[pl.ds(wbase, rows_per_w)], hix, sem_i),
            pltpu.async_copy(t_hbm.at[pl.ds(wbase, rows_per_w)], tix, sem_i),
            pltpu.async_copy(r_hbm.at[pl.ds(wbase, rows_per_w)], rix, sem_i),
            pltpu.async_copy(d_hbm.at[pl.ds(wbase, rows_per_w)], dvb, sem_i),
        ]
        for cp in cps_i:
            cp.wait()

        sets = (set0, set1)
        als = (al0, al1)
        sems = (sem0, sem1)

        def fire(ci):
            s = sets[ci % 2]
            a = als[ci % 2]
            sem = sems[ci % 2]
            hslc = hix.at[pl.ds(ci * C, C)]
            tslc = tix.at[pl.ds(ci * C, C)]
            rslc = rix.at[pl.ds(ci * C, C)]
            return [
                pltpu.async_copy(eE.at[hslc], s[0], sem),
                pltpu.async_copy(eTE.at[hslc], s[1], sem),
                pltpu.async_copy(bE.at[hslc], s[2], sem),
                pltpu.async_copy(oE.at[hslc], s[3], sem),
                pltpu.async_copy(vE.at[hslc], s[4], sem),
                pltpu.async_copy(eE.at[tslc], s[5], sem),
                pltpu.async_copy(eTE.at[tslc], s[6], sem),
                pltpu.async_copy(bE.at[tslc], s[7], sem),
                pltpu.async_copy(oE.at[tslc], s[8], sem),
                pltpu.async_copy(vE.at[tslc], s[9], sem),
                pltpu.async_copy(eR.at[rslc], s[10], sem),
                pltpu.async_copy(eTR.at[rslc], s[11], sem),
                pltpu.async_copy(bR.at[rslc], s[12], sem),
                pltpu.async_copy(oR.at[rslc], s[13], sem),
                pltpu.async_copy(vR.at[rslc], s[14], sem),
                pltpu.async_copy(aE.at[hslc], a[0], sem),
                pltpu.async_copy(aE.at[tslc], a[1], sem),
                pltpu.async_copy(aR.at[rslc], a[2], sem),
            ]

        inflight = {0: fire(0)}
        for ci in range(n_chunks):
            if ci + 1 < n_chunks:
                inflight[ci + 1] = fire(ci + 1)
            for cp in inflight.pop(ci):
                cp.wait()
            s = sets[ci % 2]
            a = als[ci % 2]

            def group_body(g, _):
                off = ci * C + g * L
                rows = lax.iota(jnp.int32, L) + g * L
                d16 = dvb[pl.ds(off, L)]
                dah = d16 * a[0][pl.ds(g * L, L)]
                dat = d16 * a[1][pl.ds(g * L, L)]
                dar = d16 * a[2][pl.ds(g * L, L)]

                def jbody(j, accs):
                    new = []
                    for u in range(JU):
                        jv = jnp.full((L,), j * JU + u, jnp.int32)
                        ld = lambda k: plsc.load_gather(s[k], [rows, jv])
                        hm = ld(0) + dah * ld(1) + ld(2) * _sin2pi(ld(3) * d16)
                        tm = ld(5) + dat * ld(6) + ld(7) * _sin2pi(ld(8) * d16)
                        rm = ld(10) + dar * ld(11) + ld(12) * _sin2pi(ld(13) * d16)
                        m = hm - tm - rm
                        m2 = m * m
                        sv = ld(4) + ld(9)
                        rv = ld(14)
                        num = sv * (sv + m2) + rv * (rv + m2)
                        new.append(accs[u] + num / (rv * sv))
                    return tuple(new)

                zero = jnp.zeros((L,), jnp.float32)
                accs = lax.fori_loop(0, D // JU, jbody, (zero,) * JU)
                acc = (accs[0] + accs[1]) + (accs[2] + accs[3])
                outb[pl.ds(off, L)] = (acc - jnp.float32(2 * D)) * jnp.float32(0.25)
                return 0

            lax.fori_loop(0, C // L, group_body, 0)

        pltpu.sync_copy(outb, out_hbm.at[pl.ds(wbase, rows_per_w)])

    return score_kernel(h_i, t_i, r_i, d_f,
                        emb_E, emb_E_var, emb_R, emb_R_var, emb_TE, alpha_E1,
                        beta_E, omega_E, emb_TR, alpha_R1, beta_R, omega_R)


# speedup vs baseline: 1.1813x; 1.1197x over previous
import functools
import jax
import jax.numpy as jnp
from jax import lax
from jax.experimental import pallas as pl
from jax.experimental.pallas import tpu as pltpu
from jax.experimental.pallas import tpu_sc as plsc

D = 64
L = 16
TWO_PI = 6.283185307179586


def _sin2pi(x):
    u = lax.rem(x, jnp.float32(1.0))
    u = jnp.where(u > 0.5, u - 1.0, u)
    u = jnp.where(u < -0.5, u + 1.0, u)
    u = jnp.where(u > 0.25, 0.5 - u, u)
    u = jnp.where(u < -0.25, -0.5 - u, u)
    th = jnp.float32(TWO_PI) * u
    t2 = th * th
    p = jnp.float32(2.7557319e-06)
    p = p * t2 - jnp.float32(1.9841270e-04)
    p = p * t2 + jnp.float32(8.3333333e-03)
    p = p * t2 - jnp.float32(0.16666667)
    p = p * t2 + jnp.float32(1.0)
    return th * p


def kernel(X, emb_E, emb_E_var, emb_R, emb_R_var, emb_TE, alpha_E, beta_E,
           omega_E, emb_TR, alpha_R, beta_R, omega_R):
    B = X.shape[0]
    h_i = X[:, 0]
    t_i = X[:, 1]
    r_i = X[:, 2]
    d_f = X[:, 3].astype(jnp.float32)
    alpha_E1 = alpha_E.reshape(-1)
    NR = emb_R.shape[0]
    relcat = jnp.concatenate(
        [emb_R, emb_TR, beta_R, omega_R, emb_R_var, alpha_R,
         jnp.zeros((NR, 7), jnp.float32)], axis=1)
    RW = 328

    info = plsc.get_sparse_core_info()
    NC, NS = info.num_cores, info.num_subcores
    NW = NC * NS
    C = 64
    rows_per_w = B // NW
    n_chunks = rows_per_w // C
    JU = 4

    mesh = plsc.VectorSubcoreMesh(core_axis_name="c", subcore_axis_name="s")

    big = lambda: pltpu.VMEM((C, D), jnp.float32)
    bigset = lambda: [big() for _ in range(10)] + [pltpu.VMEM((C, RW), jnp.float32)]

    @functools.partial(
        pl.kernel,
        out_type=jax.ShapeDtypeStruct((B,), jnp.float32),
        mesh=mesh,
        compiler_params=pltpu.CompilerParams(
            needs_layout_passes=False, use_tc_tiling_on_sc=False),
        scratch_types=[
            pltpu.VMEM((rows_per_w,), jnp.int32),
            pltpu.VMEM((rows_per_w,), jnp.int32),
            pltpu.VMEM((rows_per_w,), jnp.int32),
            pltpu.VMEM((rows_per_w,), jnp.float32),
            pltpu.VMEM((rows_per_w,), jnp.float32),
            bigset(), bigset(),
            [pltpu.VMEM((C,), jnp.float32) for _ in range(2)],
            [pltpu.VMEM((C,), jnp.float32) for _ in range(2)],
            pltpu.SemaphoreType.DMA,
            pltpu.SemaphoreType.DMA,
            pltpu.SemaphoreType.DMA,
        ],
    )
    def score_kernel(h_hbm, t_hbm, r_hbm, d_hbm,
                     eE, vE, eTE, aE, bE, oE, rcat,
                     out_hbm,
                     hix, tix, rix, dvb, outb,
                     set0, set1, al0, al1,
                     sem_i, sem0, sem1):
        wid = lax.axis_index("s") * NC + lax.axis_index("c")
        wbase = pl.multiple_of(wid * rows_per_w, rows_per_w)

        cps_i = [
            pltpu.async_copy(h_hbm.at[pl.ds(wbase, rows_per_w)], hix, sem_i),
            pltpu.async_copy(t_hbm.at[pl.ds(wbase, rows_per_w)], tix, sem_i),
            pltpu.async_copy(r_hbm.at[pl.ds(wbase, rows_per_w)], rix, sem_i),
            pltpu.async_copy(d_hbm.at[pl.ds(wbase, rows_per_w)], dvb, sem_i),
        ]
        for cp in cps_i:
            cp.wait()

        sets = (set0, set1)
        als = (al0, al1)
        sems = (sem0, sem1)

        def fire(ci):
            s = sets[ci % 2]
            a = als[ci % 2]
            sem = sems[ci % 2]
            hslc = hix.at[pl.ds(ci * C, C)]
            tslc = tix.at[pl.ds(ci * C, C)]
            rslc = rix.at[pl.ds(ci * C, C)]
            return [
                pltpu.async_copy(eE.at[hslc], s[0], sem),
                pltpu.async_copy(eTE.at[hslc], s[1], sem),
                pltpu.async_copy(bE.at[hslc], s[2], sem),
                pltpu.async_copy(oE.at[hslc], s[3], sem),
                pltpu.async_copy(vE.at[hslc], s[4], sem),
                pltpu.async_copy(eE.at[tslc], s[5], sem),
                pltpu.async_copy(eTE.at[tslc], s[6], sem),
                pltpu.async_copy(bE.at[tslc], s[7], sem),
                pltpu.async_copy(oE.at[tslc], s[8], sem),
                pltpu.async_copy(vE.at[tslc], s[9], sem),
                pltpu.async_copy(rcat.at[rslc], s[10], sem),
                pltpu.async_copy(aE.at[hslc], a[0], sem),
                pltpu.async_copy(aE.at[tslc], a[1], sem),
            ]

        inflight = {0: fire(0)}
        for ci in range(n_chunks):
            if ci + 1 < n_chunks:
                inflight[ci + 1] = fire(ci + 1)
            for cp in inflight.pop(ci):
                cp.wait()
            s = sets[ci % 2]
            a = als[ci % 2]

            def group_body(g, _):
                off = ci * C + g * L
                rows = lax.iota(jnp.int32, L) + g * L
                d16 = dvb[pl.ds(off, L)]
                dah = d16 * a[0][pl.ds(g * L, L)]
                dat = d16 * a[1][pl.ds(g * L, L)]
                dar = d16 * plsc.load_gather(
                    s[10], [rows, jnp.full((L,), 5 * D, jnp.int32)])

                def jbody(j, accs):
                    new = []
                    for u in range(JU):
                        jv = jnp.full((L,), j * JU + u, jnp.int32)
                        ld = lambda k: plsc.load_gather(s[k], [rows, jv])
                        ldr = lambda k: plsc.load_gather(s[10], [rows, jv + k * D])
                        hm = ld(0) + dah * ld(1) + ld(2) * _sin2pi(ld(3) * d16)
                        tm = ld(5) + dat * ld(6) + ld(7) * _sin2pi(ld(8) * d16)
                        rm = ldr(0) + dar * ldr(1) + ldr(2) * _sin2pi(ldr(3) * d16)
                        m = hm - tm - rm
                        m2 = m * m
                        sv = ld(4) + ld(9)
                        rv = ldr(4)
                        num = sv * (sv + m2) + rv * (rv + m2)
                        new.append(accs[u] + num / (rv * sv))
                    return tuple(new)

                zero = jnp.zeros((L,), jnp.float32)
                accs = lax.fori_loop(0, D // JU, jbody, (zero,) * JU)
                acc = (accs[0] + accs[1]) + (accs[2] + accs[3])
                outb[pl.ds(off, L)] = (acc - jnp.float32(2 * D)) * jnp.float32(0.25)
                return 0

            lax.fori_loop(0, C // L, group_body, 0)

        pltpu.sync_copy(outb, out_hbm.at[pl.ds(wbase, rows_per_w)])

    return score_kernel(h_i, t_i, r_i, d_f,
                        emb_E, emb_E_var, emb_TE, alpha_E1, beta_E, omega_E,
                        relcat)
